# baseline (speedup 1.0000x reference)
import functools

import jax
import jax.numpy as jnp
from jax import lax
from jax.experimental import pallas as pl
from jax.experimental.pallas import tpu as pltpu
from jax.experimental.pallas import tpu_sc as plsc

N = 10000
N_PAD = 10240
F = 128
NC = 2
NS = 16
CH = 128
ROWS_PER_TILE = N_PAD // NS
NCH_TILE = 80


def _sc_deg(col2, w2, e_pad):
    ept = e_pad // (NC * NS)
    nch = ept // CH
    mesh = plsc.VectorSubcoreMesh(core_axis_name="c", subcore_axis_name="s")

    @functools.partial(
        pl.kernel,
        mesh=mesh,
        out_type=jax.ShapeDtypeStruct((NC, N_PAD), jnp.float32),
        scratch_types=[
            pltpu.VMEM((NCH_TILE, CH), jnp.int32),
            pltpu.VMEM((NCH_TILE, CH), jnp.float32),
            pltpu.VMEM((ROWS_PER_TILE,), jnp.float32),
            pltpu.VMEM_SHARED((N_PAD,), jnp.float32),
        ],
    )
    def k(col_hbm, w_hbm, out_hbm, cbuf, wbuf, zb, dsh):
        cid = lax.axis_index("c")
        sid = lax.axis_index("s")
        zv = jnp.zeros((16,), jnp.float32)
        for i in range(ROWS_PER_TILE // 16):
            zb[pl.ds(i * 16, 16)] = zv
        pltpu.sync_copy(zb, dsh.at[pl.ds(sid * ROWS_PER_TILE, ROWS_PER_TILE)])
        cb = (cid * NS + sid) * nch
        pltpu.sync_copy(col_hbm.at[pl.ds(cb, nch)], cbuf)
        pltpu.sync_copy(w_hbm.at[pl.ds(cb, nch)], wbuf)
        plsc.subcore_barrier()

        def chunk(q, _):
            pltpu.sync_copy(wbuf.at[q], dsh.at[cbuf.at[q]], add=True)
            return 0

        lax.fori_loop(0, nch, chunk, 0)
        plsc.subcore_barrier()
        sl = pl.ds(sid * ROWS_PER_TILE, ROWS_PER_TILE)
        pltpu.sync_copy(dsh.at[sl], out_hbm.at[cid, sl])

    return k(col2, w2)


def _tc_prep(x, p0, p1):
    bn = 400

    def body(x_ref, p0_ref, p1_ref, u_ref, dis_ref):
        deg = 1.0 + p0_ref[...] + p1_ref[...]
        dis = lax.rsqrt(deg)
        dis_ref[...] = dis
        u_ref[...] = dis * x_ref[...]

    return pl.pallas_call(
        body,
        grid=(N // bn,),
        in_specs=[
            pl.BlockSpec((bn, F), lambda i: (i, 0)),
            pl.BlockSpec((bn, 1), lambda i: (i, 0)),
            pl.BlockSpec((bn, 1), lambda i: (i, 0)),
        ],
        out_specs=[
            pl.BlockSpec((bn, F), lambda i: (i, 0)),
            pl.BlockSpec((bn, 1), lambda i: (i, 0)),
        ],
        out_shape=[
            jax.ShapeDtypeStruct((N, F), jnp.float32),
            jax.ShapeDtypeStruct((N, 1), jnp.float32),
        ],
    )(x, p0, p1)


def _sc_agg(u, row_p, col_p, w_p, e_pad):
    ept = e_pad // (NC * NS)
    nch = ept // CH
    mesh = plsc.VectorSubcoreMesh(core_axis_name="c", subcore_axis_name="s")

    @functools.partial(
        pl.kernel,
        mesh=mesh,
        out_type=jax.ShapeDtypeStruct((NC, N_PAD, F), jnp.float32),
        scratch_types=[
            pltpu.VMEM((CH,), jnp.int32),
            pltpu.VMEM((CH,), jnp.int32),
            pltpu.VMEM((CH,), jnp.float32),
            pltpu.VMEM((CH, F), jnp.float32),
            pltpu.VMEM_SHARED((N_PAD, F), jnp.float32),
            pltpu.SemaphoreType.DMA,
        ],
    )
    def k(u_hbm, row_hbm, col_hbm, w_hbm, out_hbm, ridx, cidx, wv, rows, ysh, sem):
        cid = lax.axis_index("c")
        sid = lax.axis_index("s")
        zv = jnp.zeros((16,), jnp.float32)

        def zrow(i, _):
            for j in range(F // 16):
                rows[i, pl.ds(j * 16, 16)] = zv
            return 0

        lax.fori_loop(0, CH, zrow, 0)
        for t in range(ROWS_PER_TILE // CH):
            pltpu.sync_copy(rows, ysh.at[pl.ds(sid * ROWS_PER_TILE + t * CH, CH)])
        plsc.subcore_barrier()

        base0 = (cid * NS + sid) * ept

        def chunk(cix, _):
            b = base0 + cix * CH
            pltpu.sync_copy(row_hbm.at[pl.ds(b, CH)], ridx)
            pltpu.sync_copy(col_hbm.at[pl.ds(b, CH)], cidx)
            pltpu.sync_copy(w_hbm.at[pl.ds(b, CH)], wv)
            pltpu.async_copy(u_hbm.at[ridx], rows, sem).wait()

            def gbody(g, _):
                wg = wv[pl.ds(g * 16, 16)]
                for kk in range(16):
                    ws = wg[kk]
                    i = g * 16 + kk
                    for j in range(F // 16):
                        sl = pl.ds(j * 16, 16)
                        rows[i, sl] = rows[i, sl] * ws
                return 0

            lax.fori_loop(0, CH // 16, gbody, 0)
            pltpu.sync_copy(rows, ysh.at[cidx], add=True)
            return 0

        lax.fori_loop(0, nch, chunk, 0)
        plsc.subcore_barrier()
        sl = pl.ds(sid * ROWS_PER_TILE, ROWS_PER_TILE)
        pltpu.sync_copy(ysh.at[sl], out_hbm.at[cid, sl])

    return k(u, row_p, col_p, w_p)


def _tc_dense(a0, a1, u, dis, Wz, bz, Wlzt, blz, Wh, bh, Wlht, blh):
    bn = 400
    wspec = pl.BlockSpec((F, F), lambda i: (0, 0))
    bspec = pl.BlockSpec((1, F), lambda i: (0, 0))

    def body(a0_ref, a1_ref, u_ref, dis_ref, wz, bzr, wlz, blzr, wh, bhr,
             wlh, blhr, o_ref):
        y = dis_ref[...] * (a0_ref[...] + a1_ref[...] + u_ref[...])
        gz = jnp.dot(y, wz[...], preferred_element_type=jnp.float32) + bzr[...]
        z = jax.nn.sigmoid(
            jnp.dot(gz, wlz[...], preferred_element_type=jnp.float32) + blzr[...])
        gh = jnp.dot(y, wh[...], preferred_element_type=jnp.float32) + bhr[...]
        ht = jnp.tanh(
            jnp.dot(gh, wlh[...], preferred_element_type=jnp.float32) + blhr[...])
        o_ref[...] = jnp.maximum((1.0 - z) * ht, 0.0)

    return pl.pallas_call(
        body,
        grid=(N // bn,),
        in_specs=[
            pl.BlockSpec((bn, F), lambda i: (i, 0)),
            pl.BlockSpec((bn, F), lambda i: (i, 0)),
            pl.BlockSpec((bn, F), lambda i: (i, 0)),
            pl.BlockSpec((bn, 1), lambda i: (i, 0)),
            wspec, bspec, wspec, bspec, wspec, bspec, wspec, bspec,
        ],
        out_specs=pl.BlockSpec((bn, F), lambda i: (i, 0)),
        out_shape=jax.ShapeDtypeStruct((N, F), jnp.float32),
    )(a0, a1, u, dis, Wz, bz, Wlzt, blz, Wh, bh, Wlht, blh)


def kernel(x, edge_weight, Wz, bz, Wlz, blz, Wr, br, Wlr, blr, Wh, bh, Wlh,
           blh, edge_index):
    del Wr, br, Wlr, blr
    row = edge_index[0].astype(jnp.int32)
    col = edge_index[1].astype(jnp.int32)
    w = edge_weight.astype(jnp.float32)
    e = row.shape[0]
    gran = NC * NS * CH * 2
    e_pad = ((e + gran - 1) // gran) * gran
    pad = e_pad - e
    if pad:
        row = jnp.concatenate([row, jnp.zeros((pad,), jnp.int32)])
        col = jnp.concatenate(
            [col, N + (jnp.arange(pad, dtype=jnp.int32) % (N_PAD - N))])
        w = jnp.concatenate([w, jnp.zeros((pad,), jnp.float32)])

    assert e_pad // (NC * NS * CH) == NCH_TILE
    col2 = col.reshape(-1, CH)
    w2 = w.reshape(-1, CH)
    deg_parts = _sc_deg(col2, w2, e_pad)
    p0 = deg_parts[0, :N].reshape(N, 1)
    p1 = deg_parts[1, :N].reshape(N, 1)
    u, dis = _tc_prep(x, p0, p1)
    agg = _sc_agg(u, row, col, w, e_pad)
    return _tc_dense(
        agg[0, :N], agg[1, :N], u, dis,
        Wz, bz.reshape(1, F), Wlz[:F], blz.reshape(1, F),
        Wh, bh.reshape(1, F), Wlh[:F], blh.reshape(1, F))


# speedup vs baseline: 1.2395x; 1.2395x over previous
import functools

import jax
import jax.numpy as jnp
from jax import lax
from jax.experimental import pallas as pl
from jax.experimental.pallas import tpu as pltpu
from jax.experimental.pallas import tpu_sc as plsc

N = 10000
N_PAD = 10240
F = 128
NC = 2
NS = 16
CH = 128
ROWS_PER_TILE = N_PAD // NS


def _sc_deg(col_p, w_p, e_pad):
    ept = e_pad // (NC * NS)
    nch = ept // CH
    mesh = plsc.VectorSubcoreMesh(core_axis_name="c", subcore_axis_name="s")

    @functools.partial(
        pl.kernel,
        mesh=mesh,
        out_type=jax.ShapeDtypeStruct((NC, N_PAD), jnp.float32),
        scratch_types=[
            pltpu.VMEM((CH,), jnp.int32),
            pltpu.VMEM((CH,), jnp.float32),
            pltpu.VMEM((ROWS_PER_TILE,), jnp.float32),
            pltpu.VMEM_SHARED((N_PAD,), jnp.float32),
        ],
    )
    def k(col_hbm, w_hbm, out_hbm, cidx, wv, zb, dsh):
        cid = lax.axis_index("c")
        sid = lax.axis_index("s")
        zv = jnp.zeros((16,), jnp.float32)
        for i in range(ROWS_PER_TILE // 16):
            zb[pl.ds(i * 16, 16)] = zv
        pltpu.sync_copy(zb, dsh.at[pl.ds(sid * ROWS_PER_TILE, ROWS_PER_TILE)])
        plsc.subcore_barrier()

        base0 = (cid * NS + sid) * ept

        def chunk(cix, _):
            b = base0 + cix * CH
            pltpu.sync_copy(col_hbm.at[pl.ds(b, CH)], cidx)
            pltpu.sync_copy(w_hbm.at[pl.ds(b, CH)], wv)
            pltpu.sync_copy(wv, dsh.at[cidx], add=True)
            return 0

        lax.fori_loop(0, nch, chunk, 0)
        plsc.subcore_barrier()
        sl = pl.ds(sid * ROWS_PER_TILE, ROWS_PER_TILE)
        pltpu.sync_copy(dsh.at[sl], out_hbm.at[cid, sl])

    return k(col_p, w_p)


def _tc_prep(x, p0, p1):
    bn = 400

    def body(x_ref, p0_ref, p1_ref, u_ref, dis_ref):
        deg = 1.0 + p0_ref[...] + p1_ref[...]
        dis = lax.rsqrt(deg)
        dis_ref[...] = dis
        u_ref[...] = dis * x_ref[...]

    return pl.pallas_call(
        body,
        grid=(N // bn,),
        in_specs=[
            pl.BlockSpec((bn, F), lambda i: (i, 0)),
            pl.BlockSpec((bn, 1), lambda i: (i, 0)),
            pl.BlockSpec((bn, 1), lambda i: (i, 0)),
        ],
        out_specs=[
            pl.BlockSpec((bn, F), lambda i: (i, 0)),
            pl.BlockSpec((bn, 1), lambda i: (i, 0)),
        ],
        out_shape=[
            jax.ShapeDtypeStruct((N, F), jnp.float32),
            jax.ShapeDtypeStruct((N, 1), jnp.float32),
        ],
    )(x, p0, p1)


def _sc_agg(u, row_p, col_p, w_p, e_pad):
    ept = e_pad // (NC * NS)
    nch = ept // CH
    mesh = plsc.VectorSubcoreMesh(core_axis_name="c", subcore_axis_name="s")

    @functools.partial(
        pl.kernel,
        mesh=mesh,
        out_type=jax.ShapeDtypeStruct((NC, N_PAD, F), jnp.float32),
        scratch_types=[
            pltpu.VMEM((CH,), jnp.int32),
            pltpu.VMEM((CH,), jnp.int32),
            pltpu.VMEM((CH,), jnp.float32),
            pltpu.VMEM((CH, F), jnp.float32),
            pltpu.VMEM_SHARED((N_PAD, F), jnp.float32),
            pltpu.SemaphoreType.DMA,
        ],
    )
    def k(u_hbm, row_hbm, col_hbm, w_hbm, out_hbm, ridx, cidx, wv, rows, ysh, sem):
        cid = lax.axis_index("c")
        sid = lax.axis_index("s")
        zv = jnp.zeros((16,), jnp.float32)

        def zrow(i, _):
            for j in range(F // 16):
                rows[i, pl.ds(j * 16, 16)] = zv
            return 0

        lax.fori_loop(0, CH, zrow, 0)
        for t in range(ROWS_PER_TILE // CH):
            pltpu.sync_copy(rows, ysh.at[pl.ds(sid * ROWS_PER_TILE + t * CH, CH)])
        plsc.subcore_barrier()

        base0 = (cid * NS + sid) * ept

        def chunk(cix, _):
            b = base0 + cix * CH
            pltpu.sync_copy(row_hbm.at[pl.ds(b, CH)], ridx)
            pltpu.sync_copy(col_hbm.at[pl.ds(b, CH)], cidx)
            pltpu.sync_copy(w_hbm.at[pl.ds(b, CH)], wv)
            pltpu.async_copy(u_hbm.at[ridx], rows, sem).wait()

            def gbody(g, _):
                wg = wv[pl.ds(g * 16, 16)]
                for kk in range(16):
                    ws = wg[kk]
                    i = g * 16 + kk
                    for j in range(F // 16):
                        sl = pl.ds(j * 16, 16)
                        rows[i, sl] = rows[i, sl] * ws
                return 0

            lax.fori_loop(0, CH // 16, gbody, 0)
            pltpu.sync_copy(rows, ysh.at[cidx], add=True)
            return 0

        lax.fori_loop(0, nch, chunk, 0)
        plsc.subcore_barrier()
        sl = pl.ds(sid * ROWS_PER_TILE, ROWS_PER_TILE)
        pltpu.sync_copy(ysh.at[sl], out_hbm.at[cid, sl])

    return k(u, row_p, col_p, w_p)


def _tc_dense(a0, a1, u, dis, Wz, bz, Wlzt, blz, Wh, bh, Wlht, blh):
    bn = 400
    wspec = pl.BlockSpec((F, F), lambda i: (0, 0))
    bspec = pl.BlockSpec((1, F), lambda i: (0, 0))

    def body(a0_ref, a1_ref, u_ref, dis_ref, wz, bzr, wlz, blzr, wh, bhr,
             wlh, blhr, o_ref):
        y = dis_ref[...] * (a0_ref[...] + a1_ref[...] + u_ref[...])
        gz = jnp.dot(y, wz[...], preferred_element_type=jnp.float32) + bzr[...]
        z = jax.nn.sigmoid(
            jnp.dot(gz, wlz[...], preferred_element_type=jnp.float32) + blzr[...])
        gh = jnp.dot(y, wh[...], preferred_element_type=jnp.float32) + bhr[...]
        ht = jnp.tanh(
            jnp.dot(gh, wlh[...], preferred_element_type=jnp.float32) + blhr[...])
        o_ref[...] = jnp.maximum((1.0 - z) * ht, 0.0)

    return pl.pallas_call(
        body,
        grid=(N // bn,),
        in_specs=[
            pl.BlockSpec((bn, F), lambda i: (i, 0)),
            pl.BlockSpec((bn, F), lambda i: (i, 0)),
            pl.BlockSpec((bn, F), lambda i: (i, 0)),
            pl.BlockSpec((bn, 1), lambda i: (i, 0)),
            wspec, bspec, wspec, bspec, wspec, bspec, wspec, bspec,
        ],
        out_specs=pl.BlockSpec((bn, F), lambda i: (i, 0)),
        out_shape=jax.ShapeDtypeStruct((N, F), jnp.float32),
    )(a0, a1, u, dis, Wz, bz, Wlzt, blz, Wh, bh, Wlht, blh)


def kernel(x, edge_weight, Wz, bz, Wlz, blz, Wr, br, Wlr, blr, Wh, bh, Wlh,
           blh, edge_index):
    del Wr, br, Wlr, blr
    row = edge_index[0].astype(jnp.int32)
    col = edge_index[1].astype(jnp.int32)
    w = edge_weight.astype(jnp.float32)
    e = row.shape[0]
    gran = NC * NS * CH
    e_pad = ((e + gran - 1) // gran) * gran
    pad = e_pad - e
    if pad:
        row = jnp.concatenate([row, jnp.zeros((pad,), jnp.int32)])
        col = jnp.concatenate(
            [col, N + (jnp.arange(pad, dtype=jnp.int32) % (N_PAD - N))])
        w = jnp.concatenate([w, jnp.zeros((pad,), jnp.float32)])

    deg_parts = _sc_deg(col, w, e_pad)
    p0 = deg_parts[0, :N].reshape(N, 1)
    p1 = deg_parts[1, :N].reshape(N, 1)
    u, dis = _tc_prep(x, p0, p1)
    agg = _sc_agg(u, row, col, w, e_pad)
    return _tc_dense(
        agg[0, :N], agg[1, :N], u, dis,
        Wz, bz.reshape(1, F), Wlz[:F], blz.reshape(1, F),
        Wh, bh.reshape(1, F), Wlh[:F], blh.reshape(1, F))
